# baseline (speedup 1.0000x reference)
import functools

import jax
import jax.numpy as jnp
import numpy as np
from jax.experimental import pallas as pl
from jax.experimental.pallas import tpu as pltpu
from jax.sharding import Mesh, PartitionSpec as P


def _norm_kernel(ebt_ref, u_ref):
    e = ebt_ref[...].astype(jnp.float32)
    u_ref[...] = jnp.sum(e * e, axis=0, keepdims=True) * 0.5


def _loss_kernel(ebh_ref, ebt_ref, rowmeta_ref, colmeta_ref, out_ref, *, bn):
    j = pl.program_id(1)
    ei = ebh_ref[...]
    ejt = ebt_ref[:, pl.ds(j * bn, bn)]
    g = jnp.dot(ei, ejt, preferred_element_type=jnp.float32)

    li = rowmeta_ref[:, 0:1]
    ui = rowmeta_ref[:, 1:2]
    lj = colmeta_ref[0:1, pl.ds(j * bn, bn)]
    uj = colmeta_ref[1:2, pl.ds(j * bn, bn)]

    mse = (ui + uj) - g
    val = jnp.where(li == lj, mse, jnp.maximum(1.0 - mse, 0.0))

    @pl.when(j == 0)
    def _init():
        out_ref[...] = jnp.zeros_like(out_ref)

    out_ref[...] += jnp.sum(val, axis=0, keepdims=True)[None]


def _per_device(ebh, ebt, labf, *, bm, bn, n_shards):
    bh, d = ebh.shape
    b = ebt.shape[1]

    nbc = 2048
    u_row = pl.pallas_call(
        _norm_kernel,
        grid=(b // nbc,),
        in_specs=[pl.BlockSpec((d, nbc), lambda c: (0, c))],
        out_specs=pl.BlockSpec((1, nbc), lambda c: (0, c)),
        out_shape=jax.ShapeDtypeStruct((1, b), jnp.float32),
    )(ebt)

    colmeta = jnp.concatenate([labf[None, :], u_row], axis=0)
    didx = jax.lax.axis_index("x")
    lab_h = jax.lax.dynamic_slice_in_dim(labf, didx * bh, bh)
    u_h = jax.lax.dynamic_slice_in_dim(u_row[0], didx * bh, bh)
    rowmeta = jnp.stack([lab_h, u_h], axis=1)

    partial = pl.pallas_call(
        functools.partial(_loss_kernel, bn=bn),
        grid=(bh // bm, b // bn),
        in_specs=[
            pl.BlockSpec((bm, d), lambda i, j: (i, 0)),
            pl.BlockSpec((d, b), lambda i, j: (0, 0)),
            pl.BlockSpec((bm, 2), lambda i, j: (i, 0)),
            pl.BlockSpec((2, b), lambda i, j: (0, 0)),
        ],
        out_specs=pl.BlockSpec((1, 1, bn), lambda i, j: (i, 0, 0)),
        out_shape=jax.ShapeDtypeStruct((bh // bm, 1, bn), jnp.float32),
        compiler_params=pltpu.CompilerParams(
            dimension_semantics=("arbitrary", "arbitrary"),
        ),
    )(ebh, ebt, rowmeta, colmeta)
    return partial


def kernel(embeddings, labels):
    B, D = embeddings.shape
    BM = 512
    BN = 512

    devs = jax.devices()
    n_shards = 2 if len(devs) >= 2 else 1
    mesh = Mesh(np.array(devs[:n_shards]), ("x",))

    scale = 1.0 / jnp.sqrt(jnp.float32(D / 2))
    eb = (embeddings * scale).astype(jnp.bfloat16)
    ebt = eb.T
    labf = labels.astype(jnp.float32)

    fn = jax.shard_map(
        functools.partial(_per_device, bm=BM, bn=BN, n_shards=n_shards),
        mesh=mesh,
        in_specs=(P("x", None), P(None, None), P(None)),
        out_specs=P("x", None, None),
        check_vma=False,
    )
    partial = fn(eb, ebt, labf)

    total = jnp.sum(partial)
    return total / (2.0 * B * (B - 1))


# speedup vs baseline: 2.3792x; 2.3792x over previous
import functools

import jax
import jax.numpy as jnp
from jax.experimental import pallas as pl
from jax.experimental.pallas import tpu as pltpu


def _norm_kernel(ebt_ref, u_ref):
    e = ebt_ref[...].astype(jnp.float32)
    u_ref[...] = jnp.sum(e * e, axis=0, keepdims=True) * 0.5


def _loss_kernel(eb_ref, ebt_ref, rowmeta_ref, colmeta_ref, out_ref, *, bn):
    j = pl.program_id(1)
    ei = eb_ref[...]
    ejt = ebt_ref[:, pl.ds(j * bn, bn)]
    g = jnp.dot(ei, ejt, preferred_element_type=jnp.float32)

    li = rowmeta_ref[:, 0:1]
    ui = rowmeta_ref[:, 1:2]
    lj = colmeta_ref[0:1, pl.ds(j * bn, bn)]
    uj = colmeta_ref[1:2, pl.ds(j * bn, bn)]

    mse = (ui + uj) - g
    val = jnp.where(li == lj, mse, jnp.maximum(1.0 - mse, 0.0))

    @pl.when(j == 0)
    def _init():
        out_ref[...] = jnp.zeros_like(out_ref)

    out_ref[...] += jnp.sum(val, axis=0, keepdims=True)[None]


def kernel(embeddings, labels):
    B, D = embeddings.shape
    BM = 512
    BN = 512

    scale = 1.0 / jnp.sqrt(jnp.float32(D / 2))
    eb = (embeddings * scale).astype(jnp.bfloat16)
    ebt = eb.T
    labf = labels.astype(jnp.float32)

    NBC = 2048
    u_row = pl.pallas_call(
        _norm_kernel,
        grid=(B // NBC,),
        in_specs=[pl.BlockSpec((D, NBC), lambda c: (0, c))],
        out_specs=pl.BlockSpec((1, NBC), lambda c: (0, c)),
        out_shape=jax.ShapeDtypeStruct((1, B), jnp.float32),
    )(ebt)

    colmeta = jnp.concatenate([labf[None, :], u_row], axis=0)
    rowmeta = jnp.stack([labf, u_row[0]], axis=1)

    partial = pl.pallas_call(
        functools.partial(_loss_kernel, bn=BN),
        grid=(B // BM, B // BN),
        in_specs=[
            pl.BlockSpec((BM, D), lambda i, j: (i, 0)),
            pl.BlockSpec((D, B), lambda i, j: (0, 0)),
            pl.BlockSpec((BM, 2), lambda i, j: (i, 0)),
            pl.BlockSpec((2, B), lambda i, j: (0, 0)),
        ],
        out_specs=pl.BlockSpec((1, 1, BN), lambda i, j: (i, 0, 0)),
        out_shape=jax.ShapeDtypeStruct((B // BM, 1, BN), jnp.float32),
        compiler_params=pltpu.CompilerParams(
            dimension_semantics=("arbitrary", "arbitrary"),
        ),
    )(eb, ebt, rowmeta, colmeta)

    total = jnp.sum(partial)
    return total / (2.0 * B * (B - 1))


# speedup vs baseline: 3.6337x; 1.5273x over previous
import functools

import jax
import jax.numpy as jnp
import numpy as np
from jax.experimental import pallas as pl
from jax.experimental.pallas import tpu as pltpu


def _norm_kernel(ebt_ref, u_ref):
    e = ebt_ref[...].astype(jnp.float32)
    u_ref[...] = jnp.sum(e * e, axis=0, keepdims=True) * 0.5


def _loss_kernel(meta_ref, ei_ref, ebt_ref, rowmeta_ref, colmeta_ref, out_ref,
                 *, bm, bn):
    t = pl.program_id(0)
    bi = meta_ref[0, t]
    bj = meta_ref[1, t]
    col = pl.multiple_of(bj * bn, bn)

    ei = ei_ref[...]
    ejt = ebt_ref[:, pl.ds(col, bn)]
    g = jnp.dot(ei, ejt, preferred_element_type=jnp.float32)

    li = rowmeta_ref[:, 0:1]
    ui = rowmeta_ref[:, 1:2]
    lj = colmeta_ref[0:1, pl.ds(col, bn)]
    uj = colmeta_ref[1:2, pl.ds(col, bn)]

    mse = (ui + uj) - g
    val = jnp.where(li == lj, mse, jnp.maximum(1.0 - mse, 0.0))

    part = jnp.sum(val.reshape(bm // 8, 8, bn), axis=0)
    w = jnp.where(bi == bj, 1.0, 2.0)

    @pl.when(t == 0)
    def _init():
        out_ref[...] = jnp.zeros_like(out_ref)

    out_ref[...] += part * w


def kernel(embeddings, labels):
    B, D = embeddings.shape
    BM = 512
    BN = 512
    nb = B // BM

    scale = 1.0 / jnp.sqrt(jnp.float32(D / 2))
    eb = (embeddings * scale).astype(jnp.bfloat16)
    ebt = eb.T
    labf = labels.astype(jnp.float32)

    NBC = 2048
    u_row = pl.pallas_call(
        _norm_kernel,
        grid=(B // NBC,),
        in_specs=[pl.BlockSpec((D, NBC), lambda c: (0, c))],
        out_specs=pl.BlockSpec((1, NBC), lambda c: (0, c)),
        out_shape=jax.ShapeDtypeStruct((1, B), jnp.float32),
    )(ebt)

    colmeta = jnp.concatenate([labf[None, :], u_row], axis=0)
    rowmeta = jnp.stack([labf, u_row[0]], axis=1)

    pairs = np.array([(i, j) for i in range(nb) for j in range(i, nb)],
                     dtype=np.int32).T
    T = pairs.shape[1]
    meta = jnp.asarray(pairs)

    acc = pl.pallas_call(
        functools.partial(_loss_kernel, bm=BM, bn=BN),
        grid_spec=pltpu.PrefetchScalarGridSpec(
            num_scalar_prefetch=1,
            grid=(T,),
            in_specs=[
                pl.BlockSpec((BM, D), lambda t, m: (m[0, t], 0)),
                pl.BlockSpec((D, B), lambda t, m: (0, 0)),
                pl.BlockSpec((BM, 2), lambda t, m: (m[0, t], 0)),
                pl.BlockSpec((2, B), lambda t, m: (0, 0)),
            ],
            out_specs=pl.BlockSpec((8, BN), lambda t, m: (0, 0)),
        ),
        out_shape=jax.ShapeDtypeStruct((8, BN), jnp.float32),
        compiler_params=pltpu.CompilerParams(
            dimension_semantics=("arbitrary",),
        ),
    )(meta, eb, ebt, rowmeta, colmeta)

    total = jnp.sum(acc)
    return total / (2.0 * B * (B - 1))


# speedup vs baseline: 4.2261x; 1.1630x over previous
import functools

import jax
import jax.numpy as jnp
import numpy as np
from jax.experimental import pallas as pl
from jax.experimental.pallas import tpu as pltpu


def _norm_kernel(ebt_ref, u_ref):
    e = ebt_ref[...].astype(jnp.float32)
    u_ref[...] = jnp.sum(e * e, axis=0, keepdims=True) * 0.5


def _loss_kernel(m_ref, e0_ref, e1_ref, ebt_ref, rm0_ref, rm1_ref, cm_ref,
                 out_ref, ga_ref, gb_ref, *, bm, bn):
    s = pl.program_id(0)
    k0 = 2 * s

    @pl.when(s == 0)
    def _init():
        ga_ref[...] = jnp.zeros_like(ga_ref)
        gb_ref[...] = jnp.zeros_like(gb_ref)
        out_ref[...] = jnp.zeros_like(out_ref)

    def epilogue(g_ref, rm_ref, k):
        bj = m_ref[1, k]
        w = m_ref[2, k].astype(jnp.float32)
        col = pl.multiple_of(bj * bn, bn)
        li = rm_ref[:, 0:1]
        ui = rm_ref[:, 1:2]
        lj = cm_ref[0:1, pl.ds(col, bn)]
        uj = cm_ref[1:2, pl.ds(col, bn)]
        g = g_ref[...]
        mse = (ui + uj) - g
        val = jnp.where(li == lj, mse, jnp.maximum(1.0 - mse, 0.0))
        part = jnp.sum(val.reshape(bm // 8, 8, bn), axis=0)
        out_ref[...] += part * w

    def matmul(g_ref, e_ref, k):
        bj = m_ref[1, k]
        col = pl.multiple_of(bj * bn, bn)
        g_ref[...] = jnp.dot(e_ref[...], ebt_ref[:, pl.ds(col, bn)],
                             preferred_element_type=jnp.float32)

    epilogue(ga_ref, rm0_ref, k0)
    epilogue(gb_ref, rm1_ref, k0 + 1)
    matmul(ga_ref, e0_ref, k0 + 2)
    matmul(gb_ref, e1_ref, k0 + 3)


def kernel(embeddings, labels):
    B, D = embeddings.shape
    BM = 512
    BN = 512
    nb = B // BM

    scale = 1.0 / jnp.sqrt(jnp.float32(D / 2))
    eb = (embeddings * scale).astype(jnp.bfloat16)
    ebt = eb.T
    labf = labels.astype(jnp.float32)

    NBC = 2048
    u_row = pl.pallas_call(
        _norm_kernel,
        grid=(B // NBC,),
        in_specs=[pl.BlockSpec((D, NBC), lambda c: (0, c))],
        out_specs=pl.BlockSpec((1, NBC), lambda c: (0, c)),
        out_shape=jax.ShapeDtypeStruct((1, B), jnp.float32),
    )(ebt)

    colmeta = jnp.concatenate([labf[None, :], u_row], axis=0)
    rowmeta = jnp.stack([labf, u_row[0]], axis=1)

    pairs = [(i, j, 1 if i == j else 2)
             for i in range(nb) for j in range(i, nb)]
    L = len(pairs)
    slots = [(0, 0, 0)] * 2 + pairs + [(0, 0, 0)] * 2
    meta = jnp.asarray(np.array(slots, dtype=np.int32).T)
    S = L // 2 + 1

    acc = pl.pallas_call(
        functools.partial(_loss_kernel, bm=BM, bn=BN),
        grid_spec=pltpu.PrefetchScalarGridSpec(
            num_scalar_prefetch=1,
            grid=(S,),
            in_specs=[
                pl.BlockSpec((BM, D), lambda s, m: (m[0, 2 * s + 2], 0)),
                pl.BlockSpec((BM, D), lambda s, m: (m[0, 2 * s + 3], 0)),
                pl.BlockSpec((D, B), lambda s, m: (0, 0)),
                pl.BlockSpec((BM, 2), lambda s, m: (m[0, 2 * s], 0)),
                pl.BlockSpec((BM, 2), lambda s, m: (m[0, 2 * s + 1], 0)),
                pl.BlockSpec((2, B), lambda s, m: (0, 0)),
            ],
            out_specs=pl.BlockSpec((8, BN), lambda s, m: (0, 0)),
            scratch_shapes=[
                pltpu.VMEM((BM, BN), jnp.float32),
                pltpu.VMEM((BM, BN), jnp.float32),
            ],
        ),
        out_shape=jax.ShapeDtypeStruct((8, BN), jnp.float32),
        compiler_params=pltpu.CompilerParams(
            dimension_semantics=("arbitrary",),
        ),
    )(meta, eb, eb, ebt, rowmeta, rowmeta, colmeta)

    total = jnp.sum(acc)
    return total / (2.0 * B * (B - 1))


# speedup vs baseline: 5.8491x; 1.3840x over previous
import functools

import jax
import jax.numpy as jnp
import numpy as np
from jax.experimental import pallas as pl
from jax.experimental.pallas import tpu as pltpu


def _prep_kernel(e_ref, eb_ref, u_ref, *, scale):
    e = e_ref[...] * scale
    eb_ref[...] = e.astype(jnp.bfloat16)
    u_ref[...] = 0.5 * jnp.sum(e * e, axis=1, keepdims=True)


def _loss_kernel(m_ref, eb_ref, rm0_ref, rm1_ref, cm_ref,
                 out_ref, ga_ref, gb_ref, *, bm, bn):
    s = pl.program_id(0)
    k0 = 2 * s

    @pl.when(s == 0)
    def _init():
        ga_ref[...] = jnp.zeros_like(ga_ref)
        gb_ref[...] = jnp.zeros_like(gb_ref)
        out_ref[...] = jnp.zeros_like(out_ref)

    def epilogue(g_ref, rm_ref, k):
        bj = m_ref[1, k]
        w = m_ref[2, k].astype(jnp.float32)
        col = pl.multiple_of(bj * bn, bn)
        li = rm_ref[:, 0:1]
        ui = rm_ref[:, 1:2]
        lj = cm_ref[0:1, pl.ds(col, bn)]
        uj = cm_ref[1:2, pl.ds(col, bn)]
        g = g_ref[...]
        mse = (ui + uj) - g
        val = jnp.where(li == lj, mse, jnp.maximum(1.0 - mse, 0.0))
        part = jnp.sum(val.reshape(bm // 8, 8, bn), axis=0)
        out_ref[...] += part * w

    def matmul(g_ref, k):
        bi = m_ref[0, k]
        bj = m_ref[1, k]
        row = pl.multiple_of(bi * bm, bm)
        col = pl.multiple_of(bj * bn, bn)
        ei = eb_ref[pl.ds(row, bm), :]
        ej = eb_ref[pl.ds(col, bn), :]
        g_ref[...] = jax.lax.dot_general(
            ei, ej, (((1,), (1,)), ((), ())),
            preferred_element_type=jnp.float32)

    epilogue(ga_ref, rm0_ref, k0)
    epilogue(gb_ref, rm1_ref, k0 + 1)
    matmul(ga_ref, k0 + 2)
    matmul(gb_ref, k0 + 3)


def kernel(embeddings, labels):
    B, D = embeddings.shape
    BM = 512
    BN = 512
    nb = B // BM

    labf = labels.astype(jnp.float32)
    scale = 1.0 / float(np.sqrt(D / 2))

    NB = 1024
    eb, u_col = pl.pallas_call(
        functools.partial(_prep_kernel, scale=scale),
        grid=(B // NB,),
        in_specs=[pl.BlockSpec((NB, D), lambda i: (i, 0))],
        out_specs=[
            pl.BlockSpec((NB, D), lambda i: (i, 0)),
            pl.BlockSpec((NB, 1), lambda i: (i, 0)),
        ],
        out_shape=[
            jax.ShapeDtypeStruct((B, D), jnp.bfloat16),
            jax.ShapeDtypeStruct((B, 1), jnp.float32),
        ],
    )(embeddings)

    u = u_col[:, 0]
    colmeta = jnp.stack([labf, u], axis=0)
    rowmeta = jnp.stack([labf, u], axis=1)

    pairs = [(i, j, 1 if i == j else 2)
             for i in range(nb) for j in range(i, nb)]
    L = len(pairs)
    slots = [(0, 0, 0)] * 2 + pairs + [(0, 0, 0)] * 2
    meta = jnp.asarray(np.array(slots, dtype=np.int32).T)
    S = L // 2 + 1

    acc = pl.pallas_call(
        functools.partial(_loss_kernel, bm=BM, bn=BN),
        grid_spec=pltpu.PrefetchScalarGridSpec(
            num_scalar_prefetch=1,
            grid=(S,),
            in_specs=[
                pl.BlockSpec((B, D), lambda s, m: (0, 0)),
                pl.BlockSpec((BM, 2), lambda s, m: (m[0, 2 * s], 0)),
                pl.BlockSpec((BM, 2), lambda s, m: (m[0, 2 * s + 1], 0)),
                pl.BlockSpec((2, B), lambda s, m: (0, 0)),
            ],
            out_specs=pl.BlockSpec((8, BN), lambda s, m: (0, 0)),
            scratch_shapes=[
                pltpu.VMEM((BM, BN), jnp.float32),
                pltpu.VMEM((BM, BN), jnp.float32),
            ],
        ),
        out_shape=jax.ShapeDtypeStruct((8, BN), jnp.float32),
        compiler_params=pltpu.CompilerParams(
            dimension_semantics=("arbitrary",),
        ),
    )(meta, eb, rowmeta, rowmeta, colmeta)

    total = jnp.sum(acc)
    return total / (2.0 * B * (B - 1))
